# baseline (speedup 1.0000x reference)
import functools

import jax
import jax.numpy as jnp
from jax import lax
from jax.experimental import pallas as pl
from jax.experimental.pallas import tpu as pltpu
from jax.experimental.pallas import tpu_sc as plsc

N = 8388608
NC, NS, L = 2, 16, 16
NW = NC * NS
NPW = N // NW
SEG = 512
GRP = 2
CHUNK = GRP * L * SEG
NCHUNK = NPW // CHUNK
SPC = GRP * L
SPW = NCHUNK * SPC
NSEG = N // SEG
UNROLL = 4

TR = 2048
TG = NSEG // TR

_mesh = plsc.VectorSubcoreMesh(core_axis_name="c", subcore_axis_name="s")
_params = pltpu.CompilerParams(needs_layout_passes=False)


def _offsets_body(x_ref, o_ref, rs_ref):
    i = pl.program_id(0)
    x = jnp.reshape(x_ref[...], (TR, SEG))
    rs = jnp.sum(x, axis=1)
    rs_ref[pl.ds(i * (TR // 128), TR // 128), :] = jnp.reshape(
        rs, (TR // 128, 128))

    @pl.when(i == TG - 1)
    def _():
        m = rs_ref[...]
        ri = lax.broadcasted_iota(jnp.int32, (128, 128), 0)
        ci = lax.broadcasted_iota(jnp.int32, (128, 128), 1)
        u = (ri <= ci).astype(jnp.float32)
        sl = (ci < ri).astype(jnp.float32)
        incl = lax.dot_general(m, u, (((1,), (0,)), ((), ())),
                               preferred_element_type=jnp.float32,
                               precision=lax.Precision.HIGHEST)
        rowoff = lax.dot_general(sl, incl[:, 127:128],
                                 (((1,), (0,)), ((), ())),
                                 preferred_element_type=jnp.float32,
                                 precision=lax.Precision.HIGHEST)
        o_ref[...] = incl - m + rowoff


def _seg_offsets(x1):
    return pl.pallas_call(
        _offsets_body,
        grid=(TG,),
        in_specs=[pl.BlockSpec((TR * SEG,), lambda i: (i,))],
        out_specs=pl.BlockSpec((128, 128), lambda i: (0, 0)),
        out_shape=jax.ShapeDtypeStruct((128, 128), jnp.float32),
        scratch_shapes=[pltpu.VMEM((128, 128), jnp.float32)],
    )(x1)


@functools.partial(
    pl.kernel,
    mesh=_mesh,
    out_type=jax.ShapeDtypeStruct((N,), jnp.float32),
    compiler_params=_params,
    scratch_types=[
        pltpu.VMEM((SPW,), jnp.float32),
        pltpu.VMEM((CHUNK,), jnp.float32),
        pltpu.VMEM((CHUNK,), jnp.float32),
        pltpu.VMEM((CHUNK,), jnp.float32),
        pltpu.VMEM((CHUNK,), jnp.float32),
        pltpu.SemaphoreType.DMA,
        pltpu.SemaphoreType.DMA,
        pltpu.SemaphoreType.DMA,
        pltpu.SemaphoreType.DMA,
    ],
)
def _scan_apply(x_hbm, offs_hbm, out_hbm,
                offs_v, ibuf0, ibuf1, obuf0, obuf1,
                isem0, isem1, osem0, osem1):
    wid = lax.axis_index("s") * NC + lax.axis_index("c")
    base = wid * NPW
    ibufs = (ibuf0, ibuf1)
    obufs = (obuf0, obuf1)
    isems = (isem0, isem1)
    osems = (osem0, osem1)

    pltpu.sync_copy(offs_hbm.at[pl.ds(wid * SPW, SPW)], offs_v)

    lane = lax.iota(jnp.int32, L)
    idx0 = tuple(lane * (SEG - 1) + g * (L * SEG) for g in range(GRP))

    ih = [None, None]
    oh = [None, None]
    ih[0] = pltpu.async_copy(x_hbm.at[pl.ds(base, CHUNK)], ibuf0, isem0)
    ih[1] = pltpu.async_copy(x_hbm.at[pl.ds(base + CHUNK, CHUNK)],
                             ibuf1, isem1)
    for c in range(NCHUNK):
        b = c % 2
        ih[b].wait()
        if oh[b] is not None:
            oh[b].wait()
        ibuf = ibufs[b]
        obuf = obufs[b]
        offs = tuple(offs_v[pl.ds(c * SPC + g * L, L)] for g in range(GRP))

        @plsc.parallel_loop(0, SEG + L - 1, unroll=UNROLL,
                            carry=(offs, idx0, -lane))
        def t_body(t, ai):
            accs, idxs, d = ai
            mask = (d >= 0) & (d < SEG)
            new_accs = []
            for g in range(GRP):
                v = plsc.load_gather(ibuf, [idxs[g]], mask=mask)
                a = accs[g] + jnp.where(mask, v, jnp.float32(0.0))
                plsc.store_scatter(obuf, [idxs[g]], a, mask=mask)
                new_accs.append(a)
            return tuple(new_accs), tuple(i + 1 for i in idxs), d + 1

        del t_body
        oh[b] = pltpu.async_copy(
            obuf, out_hbm.at[pl.ds(base + c * CHUNK, CHUNK)], osems[b])
        if c + 2 < NCHUNK:
            ih[b] = pltpu.async_copy(
                x_hbm.at[pl.ds(base + (c + 2) * CHUNK, CHUNK)],
                ibuf, isems[b])

    oh[0].wait()
    oh[1].wait()


def kernel(input_array):
    offs = _seg_offsets(input_array)
    return _scan_apply(input_array, offs.reshape(NSEG))


# speedup vs baseline: 1.0049x; 1.0049x over previous
import functools

import jax
import jax.numpy as jnp
from jax import lax
from jax.experimental import pallas as pl
from jax.experimental.pallas import tpu as pltpu
from jax.experimental.pallas import tpu_sc as plsc

N = 8388608
NC, NS, L = 2, 16, 16
NW = NC * NS
NPW = N // NW
SEG = 512
GRP = 4
CHUNK = GRP * L * SEG
NCHUNK = NPW // CHUNK
SPC = GRP * L
SPW = NCHUNK * SPC
NSEG = N // SEG
UNROLL = 4

TR = 2048
TG = NSEG // TR

_mesh = plsc.VectorSubcoreMesh(core_axis_name="c", subcore_axis_name="s")
_params = pltpu.CompilerParams(needs_layout_passes=False)


def _offsets_body(x_ref, o_ref, rs_ref):
    i = pl.program_id(0)
    x = jnp.reshape(x_ref[...], (TR, SEG))
    rs = jnp.sum(x, axis=1)
    rs_ref[pl.ds(i * (TR // 128), TR // 128), :] = jnp.reshape(
        rs, (TR // 128, 128))

    @pl.when(i == TG - 1)
    def _():
        m = rs_ref[...]
        ri = lax.broadcasted_iota(jnp.int32, (128, 128), 0)
        ci = lax.broadcasted_iota(jnp.int32, (128, 128), 1)
        u = (ri <= ci).astype(jnp.float32)
        sl = (ci < ri).astype(jnp.float32)
        incl = lax.dot_general(m, u, (((1,), (0,)), ((), ())),
                               preferred_element_type=jnp.float32,
                               precision=lax.Precision.HIGHEST)
        rowoff = lax.dot_general(sl, incl[:, 127:128],
                                 (((1,), (0,)), ((), ())),
                                 preferred_element_type=jnp.float32,
                                 precision=lax.Precision.HIGHEST)
        o_ref[...] = incl - m + rowoff


def _seg_offsets(x1):
    return pl.pallas_call(
        _offsets_body,
        grid=(TG,),
        in_specs=[pl.BlockSpec((TR * SEG,), lambda i: (i,))],
        out_specs=pl.BlockSpec((128, 128), lambda i: (0, 0)),
        out_shape=jax.ShapeDtypeStruct((128, 128), jnp.float32),
        scratch_shapes=[pltpu.VMEM((128, 128), jnp.float32)],
    )(x1)


@functools.partial(
    pl.kernel,
    mesh=_mesh,
    out_type=jax.ShapeDtypeStruct((N,), jnp.float32),
    compiler_params=_params,
    scratch_types=[
        pltpu.VMEM((SPW,), jnp.float32),
        pltpu.VMEM((CHUNK,), jnp.float32),
        pltpu.VMEM((CHUNK,), jnp.float32),
        pltpu.SemaphoreType.DMA,
        pltpu.SemaphoreType.DMA,
        pltpu.SemaphoreType.DMA,
        pltpu.SemaphoreType.DMA,
    ],
)
def _scan_apply(x_hbm, offs_hbm, out_hbm,
                offs_v, buf0, buf1, isem0, isem1, osem0, osem1):
    wid = lax.axis_index("s") * NC + lax.axis_index("c")
    base = wid * NPW
    bufs = (buf0, buf1)
    isems = (isem0, isem1)
    osems = (osem0, osem1)

    pltpu.sync_copy(offs_hbm.at[pl.ds(wid * SPW, SPW)], offs_v)

    lane = lax.iota(jnp.int32, L)
    idx0 = tuple(lane * (SEG - 1) + g * (L * SEG) for g in range(GRP))

    ih = [None, None]
    oh = [None, None]
    ih[0] = pltpu.async_copy(x_hbm.at[pl.ds(base, CHUNK)], buf0, isem0)
    for c in range(NCHUNK):
        b = c % 2
        if c + 1 < NCHUNK:
            if oh[1 - b] is not None:
                oh[1 - b].wait()
            ih[1 - b] = pltpu.async_copy(
                x_hbm.at[pl.ds(base + (c + 1) * CHUNK, CHUNK)],
                bufs[1 - b], isems[1 - b])
        ih[b].wait()
        buf = bufs[b]
        offs = tuple(offs_v[pl.ds(c * SPC + g * L, L)] for g in range(GRP))

        @plsc.parallel_loop(0, SEG + L - 1, unroll=UNROLL,
                            carry=(offs, idx0, -lane))
        def t_body(t, ai):
            accs, idxs, d = ai
            mask = (d >= 0) & (d < SEG)
            new_accs = []
            for g in range(GRP):
                v = plsc.load_gather(buf, [idxs[g]], mask=mask)
                a = accs[g] + jnp.where(mask, v, jnp.float32(0.0))
                plsc.store_scatter(buf, [idxs[g]], a, mask=mask)
                new_accs.append(a)
            return tuple(new_accs), tuple(i + 1 for i in idxs), d + 1

        del t_body
        oh[b] = pltpu.async_copy(
            buf, out_hbm.at[pl.ds(base + c * CHUNK, CHUNK)], osems[b])

    oh[0].wait()
    oh[1].wait()


def kernel(input_array):
    offs = _seg_offsets(input_array)
    return _scan_apply(input_array, offs.reshape(NSEG))


# speedup vs baseline: 1.0050x; 1.0001x over previous
import functools

import jax
import jax.numpy as jnp
from jax import lax
from jax.experimental import pallas as pl
from jax.experimental.pallas import tpu as pltpu
from jax.experimental.pallas import tpu_sc as plsc

N = 8388608
NC, NS, L = 2, 16, 16
NW = NC * NS
NPW = N // NW
SEG = 512
GRP = 4
CHUNK = GRP * L * SEG
NCHUNK = NPW // CHUNK
SPC = GRP * L
SPW = NCHUNK * SPC
NSEG = N // SEG
UNROLL = 4

TR = 2048
TG = NSEG // TR

_mesh = plsc.VectorSubcoreMesh(core_axis_name="c", subcore_axis_name="s")
_params = pltpu.CompilerParams(needs_layout_passes=False)


def _offsets_body(x_ref, o_ref, rs_ref):
    i = pl.program_id(0)
    x = jnp.reshape(x_ref[...], (TR, SEG))
    rs = jnp.sum(x, axis=1)
    rs_ref[pl.ds(i * (TR // 128), TR // 128), :] = jnp.reshape(
        rs, (TR // 128, 128))

    @pl.when(i == TG - 1)
    def _():
        m = rs_ref[...]
        ri = lax.broadcasted_iota(jnp.int32, (128, 128), 0)
        ci = lax.broadcasted_iota(jnp.int32, (128, 128), 1)
        u = (ri <= ci).astype(jnp.float32)
        sl = (ci < ri).astype(jnp.float32)
        incl = lax.dot_general(m, u, (((1,), (0,)), ((), ())),
                               preferred_element_type=jnp.float32,
                               precision=lax.Precision.HIGHEST)
        rowoff = lax.dot_general(sl, incl[:, 127:128],
                                 (((1,), (0,)), ((), ())),
                                 preferred_element_type=jnp.float32,
                                 precision=lax.Precision.HIGHEST)
        o_ref[...] = incl - m + rowoff


def _seg_offsets(x1):
    return pl.pallas_call(
        _offsets_body,
        grid=(TG,),
        in_specs=[pl.BlockSpec((TR * SEG,), lambda i: (i,))],
        out_specs=pl.BlockSpec((128, 128), lambda i: (0, 0)),
        out_shape=jax.ShapeDtypeStruct((128, 128), jnp.float32),
        scratch_shapes=[pltpu.VMEM((128, 128), jnp.float32)],
    )(x1)


@functools.partial(
    pl.kernel,
    mesh=_mesh,
    out_type=jax.ShapeDtypeStruct((N,), jnp.float32),
    compiler_params=_params,
    scratch_types=[
        pltpu.VMEM((SPW // 128, 128), jnp.float32),
        pltpu.VMEM((CHUNK,), jnp.float32),
        pltpu.VMEM((CHUNK,), jnp.float32),
        pltpu.SemaphoreType.DMA,
        pltpu.SemaphoreType.DMA,
        pltpu.SemaphoreType.DMA,
        pltpu.SemaphoreType.DMA,
    ],
)
def _scan_apply(x_hbm, offs_hbm, out_hbm,
                offs_v, buf0, buf1, isem0, isem1, osem0, osem1):
    wid = lax.axis_index("s") * NC + lax.axis_index("c")
    base = wid * NPW
    bufs = (buf0, buf1)
    isems = (isem0, isem1)
    osems = (osem0, osem1)

    pltpu.sync_copy(
        offs_hbm.at[pl.ds(wid * (SPW // 128), SPW // 128), :], offs_v)

    lane = lax.iota(jnp.int32, L)
    idx0 = tuple(lane * (SEG - 1) + g * (L * SEG) for g in range(GRP))

    ih = [None, None]
    oh = [None, None]
    ih[0] = pltpu.async_copy(x_hbm.at[pl.ds(base, CHUNK)], buf0, isem0)
    for c in range(NCHUNK):
        b = c % 2
        if c + 1 < NCHUNK:
            if oh[1 - b] is not None:
                oh[1 - b].wait()
            ih[1 - b] = pltpu.async_copy(
                x_hbm.at[pl.ds(base + (c + 1) * CHUNK, CHUNK)],
                bufs[1 - b], isems[1 - b])
        ih[b].wait()
        buf = bufs[b]
        offs = tuple(
            offs_v[(c * SPC + g * L) // 128,
                   pl.ds((c * SPC + g * L) % 128, L)]
            for g in range(GRP))

        @plsc.parallel_loop(0, SEG + L - 1, unroll=UNROLL,
                            carry=(offs, idx0, -lane))
        def t_body(t, ai):
            accs, idxs, d = ai
            mask = (d >= 0) & (d < SEG)
            new_accs = []
            for g in range(GRP):
                v = plsc.load_gather(buf, [idxs[g]], mask=mask)
                a = accs[g] + jnp.where(mask, v, jnp.float32(0.0))
                plsc.store_scatter(buf, [idxs[g]], a, mask=mask)
                new_accs.append(a)
            return tuple(new_accs), tuple(i + 1 for i in idxs), d + 1

        del t_body
        oh[b] = pltpu.async_copy(
            buf, out_hbm.at[pl.ds(base + c * CHUNK, CHUNK)], osems[b])

    oh[0].wait()
    oh[1].wait()


def kernel(input_array):
    offs = _seg_offsets(input_array)
    return _scan_apply(input_array, offs)
